# baseline (speedup 1.0000x reference)
import jax
import jax.numpy as jnp
from jax.experimental import pallas as pl

SEQ = 1024
DIM = 128
KEEP = int(0.3 * SEQ * SEQ)

_MININT = -(2**31)


def _gcn_kernel(x_ref, w_ref, b_ref, out_ref):
    xs = x_ref[...]

    ew = jax.lax.dot_general(
        xs, xs, (((1,), (1,)), ((), ())),
        preferred_element_type=jnp.float32)
    ii = jax.lax.broadcasted_iota(jnp.int32, (SEQ, SEQ), 0)
    jj = jax.lax.broadcasted_iota(jnp.int32, (SEQ, SEQ), 1)
    ew = ew - jnp.where(ii == jj, 1.0, 0.0).astype(jnp.float32)

    ibits = jax.lax.bitcast_convert_type(ew, jnp.int32)
    key = jnp.where(ibits >= 0, ibits, ibits ^ 0x7FFFFFFF)

    ud = jax.lax.broadcasted_iota(jnp.int32, (128, 128), 0)
    vd = jax.lax.broadcasted_iota(jnp.int32, (128, 128), 1)
    wdg = jnp.where(vd > ud, 2.0,
                    jnp.where(vd == ud, 1.0, 0.0)).astype(jnp.float32)

    def count_cmp(cs, strict):
        total = 0.0
        for blk in range(8):
            lo = 128 * blk
            dsub = key[lo:lo + 128, lo:lo + 128]
            md = (dsub > cs) if strict else (dsub >= cs)
            total += jnp.sum(
                jnp.sum(jnp.where(md, wdg, 0.0), axis=0, keepdims=True))
            if blk < 7:
                rsub = key[lo:lo + 128, lo + 128:]
                mr = (rsub > cs) if strict else (rsub >= cs)
                total += jnp.sum(
                    jnp.sum(jnp.where(mr, 2.0, 0.0), axis=0, keepdims=True))
        return total

    def count3(cs1, cs2, cs3):
        t1 = t2 = t3 = 0.0
        for blk in range(8):
            lo = 128 * blk
            dsub = key[lo:lo + 128, lo:lo + 128]
            t1 += jnp.sum(jnp.sum(
                jnp.where(dsub >= cs1, wdg, 0.0), axis=0, keepdims=True))
            t2 += jnp.sum(jnp.sum(
                jnp.where(dsub >= cs2, wdg, 0.0), axis=0, keepdims=True))
            t3 += jnp.sum(jnp.sum(
                jnp.where(dsub >= cs3, wdg, 0.0), axis=0, keepdims=True))
            if blk < 7:
                rsub = key[lo:lo + 128, lo + 128:]
                t1 += jnp.sum(jnp.sum(
                    jnp.where(rsub >= cs1, 2.0, 0.0), axis=0, keepdims=True))
                t2 += jnp.sum(jnp.sum(
                    jnp.where(rsub >= cs2, 2.0, 0.0), axis=0, keepdims=True))
                t3 += jnp.sum(jnp.sum(
                    jnp.where(rsub >= cs3, 2.0, 0.0), axis=0, keepdims=True))
        return t1, t2, t3

    def value_round(i, obits):
        sh = 30 - 2 * i
        chi = obits | jnp.left_shift(1, sh + 1)
        clo = obits | jnp.left_shift(1, sh)
        cboth = chi | jnp.left_shift(1, sh)
        nhi, nlo, nboth = count3(
            chi ^ _MININT, clo ^ _MININT, cboth ^ _MININT)
        kf = float(KEEP)
        ob = jnp.where(nlo >= kf, clo, obits)
        return jnp.where(nhi >= kf,
                         jnp.where(nboth >= kf, cboth, chi), ob)

    obits = jax.lax.fori_loop(0, 16, value_round, jnp.int32(0))
    t_key = obits ^ _MININT

    above = key > t_key
    tie = key == t_key
    r_f = float(KEEP) - count_cmp(t_key, True)

    tie_bf = jnp.where(tie, 1.0, 0.0).astype(jnp.bfloat16)
    l_bf = jnp.where(jj < ii, 1.0, 0.0).astype(jnp.bfloat16)
    wc = jax.lax.dot_general(
        l_bf, tie_bf, (((1,), (0,)), ((), ())),
        preferred_element_type=jnp.float32)
    rc = wc[SEQ - 1:SEQ, :] + tie[SEQ - 1:SEQ, :].astype(jnp.float32)
    inc = rc
    s = 1
    while s < SEQ:
        inc = inc + jnp.concatenate(
            [jnp.zeros((1, s), jnp.float32), inc[:, :SEQ - s]], axis=1)
        s *= 2
    row_off = inc - rc

    keep_t = above | (tie & ((row_off + wc) < r_f))
    at = jnp.where(keep_t, ew, 0.0)

    deg = jnp.sum(at, axis=1, keepdims=True) + 1.0
    dis = deg ** -0.5
    dis = jnp.where(jnp.isinf(dis), 0.0, dis)

    xt = jax.lax.dot_general(
        xs, w_ref[...], (((1,), (1,)), ((), ())),
        preferred_element_type=jnp.float32)

    sx = dis * xt
    y = jax.lax.dot_general(
        at, sx, (((1,), (0,)), ((), ())),
        preferred_element_type=jnp.float32)

    out_ref[...] = dis * y + (dis * dis) * xt + b_ref[...]


def kernel(x, W, b):
    xs = x.reshape(SEQ, DIM)
    b2 = b.reshape(1, DIM)
    out = pl.pallas_call(
        _gcn_kernel,
        out_shape=jax.ShapeDtypeStruct((SEQ, DIM), jnp.float32),
    )(xs, W, b2)
    return out[None, :, :]


# speedup vs baseline: 1.1457x; 1.1457x over previous
import jax
import jax.numpy as jnp
from jax.experimental import pallas as pl

SEQ = 1024
DIM = 128
KEEP = int(0.3 * SEQ * SEQ)

_MININT = -(2**31)


def _gcn_kernel(x_ref, w_ref, b_ref, out_ref):
    xs = x_ref[...]

    ew = jax.lax.dot_general(
        xs, xs, (((1,), (1,)), ((), ())),
        preferred_element_type=jnp.float32)
    ii = jax.lax.broadcasted_iota(jnp.int32, (SEQ, SEQ), 0)
    jj = jax.lax.broadcasted_iota(jnp.int32, (SEQ, SEQ), 1)
    ew = ew - jnp.where(ii == jj, 1.0, 0.0).astype(jnp.float32)

    ibits = jax.lax.bitcast_convert_type(ew, jnp.int32)
    key = jnp.where(ibits >= 0, ibits, ibits ^ 0x7FFFFFFF)

    ud = jax.lax.broadcasted_iota(jnp.int32, (128, 128), 0)
    vd = jax.lax.broadcasted_iota(jnp.int32, (128, 128), 1)
    wdg = jnp.where(vd > ud, 2.0,
                    jnp.where(vd == ud, 1.0, 0.0)).astype(jnp.float32)

    def count_cmp(cs, strict):
        row = None
        for blk in range(8):
            lo = 128 * blk
            dsub = key[lo:lo + 128, lo:lo + 128]
            md = (dsub > cs) if strict else (dsub >= cs)
            part = jnp.sum(jnp.where(md, wdg, 0.0), axis=0, keepdims=True)
            if blk < 7:
                rsub = key[lo:lo + 128, lo + 128:]
                mr = (rsub > cs) if strict else (rsub >= cs)
                part = jnp.concatenate([part, jnp.sum(
                    jnp.where(mr, 2.0, 0.0), axis=0, keepdims=True)], axis=1)
            if lo:
                part = jnp.concatenate(
                    [jnp.zeros((1, lo), jnp.float32), part], axis=1)
            row = part if row is None else row + part
        return jnp.sum(row)

    def value_step(i, obits):
        cand = obits | jnp.left_shift(1, 31 - i)
        cnt = count_cmp(cand ^ _MININT, False)
        return jnp.where(cnt >= float(KEEP), cand, obits)

    obits = jax.lax.fori_loop(0, 32, value_step, jnp.int32(0))
    t_key = obits ^ _MININT

    above = key > t_key
    tie = key == t_key
    r_f = float(KEEP) - count_cmp(t_key, True)

    tie_bf = jnp.where(tie, 1.0, 0.0).astype(jnp.bfloat16)
    l_bf = jnp.where(jj < ii, 1.0, 0.0).astype(jnp.bfloat16)
    wc = jax.lax.dot_general(
        l_bf, tie_bf, (((1,), (0,)), ((), ())),
        preferred_element_type=jnp.float32)
    rc = wc[SEQ - 1:SEQ, :] + tie[SEQ - 1:SEQ, :].astype(jnp.float32)
    inc = rc
    s = 1
    while s < SEQ:
        inc = inc + jnp.concatenate(
            [jnp.zeros((1, s), jnp.float32), inc[:, :SEQ - s]], axis=1)
        s *= 2
    row_off = inc - rc

    keep_t = above | (tie & ((row_off + wc) < r_f))
    at = jnp.where(keep_t, ew, 0.0)

    deg = jnp.sum(at, axis=1, keepdims=True) + 1.0
    dis = deg ** -0.5
    dis = jnp.where(jnp.isinf(dis), 0.0, dis)

    xt = jax.lax.dot_general(
        xs, w_ref[...], (((1,), (1,)), ((), ())),
        preferred_element_type=jnp.float32)

    sx = dis * xt
    y = jax.lax.dot_general(
        at, sx, (((1,), (0,)), ((), ())),
        preferred_element_type=jnp.float32)

    out_ref[...] = dis * y + (dis * dis) * xt + b_ref[...]


def kernel(x, W, b):
    xs = x.reshape(SEQ, DIM)
    b2 = b.reshape(1, DIM)
    out = pl.pallas_call(
        _gcn_kernel,
        out_shape=jax.ShapeDtypeStruct((SEQ, DIM), jnp.float32),
    )(xs, W, b2)
    return out[None, :, :]


# speedup vs baseline: 1.1701x; 1.0213x over previous
import jax
import jax.numpy as jnp
from jax.experimental import pallas as pl

SEQ = 1024
DIM = 128
KEEP = int(0.3 * SEQ * SEQ)

_MININT = -(2**31)


def _gcn_kernel(x_ref, w_ref, b_ref, out_ref):
    xs = x_ref[...]

    ew = jax.lax.dot_general(
        xs, xs, (((1,), (1,)), ((), ())),
        preferred_element_type=jnp.float32)
    ii = jax.lax.broadcasted_iota(jnp.int32, (SEQ, SEQ), 0)
    jj = jax.lax.broadcasted_iota(jnp.int32, (SEQ, SEQ), 1)
    ew = ew - jnp.where(ii == jj, 1.0, 0.0).astype(jnp.float32)

    ibits = jax.lax.bitcast_convert_type(ew, jnp.int32)
    key = jnp.where(ibits >= 0, ibits, ibits ^ 0x7FFFFFFF)

    ud = jax.lax.broadcasted_iota(jnp.int32, (128, 128), 0)
    vd = jax.lax.broadcasted_iota(jnp.int32, (128, 128), 1)
    wdg = jnp.where(vd > ud, 2.0,
                    jnp.where(vd == ud, 1.0, 0.0)).astype(jnp.float32)

    def count_cmp(cs, strict):
        row = None
        for blk in range(8):
            lo = 128 * blk
            dsub = key[lo:lo + 128, lo:lo + 128]
            md = (dsub > cs) if strict else (dsub >= cs)
            part = jnp.sum(jnp.where(md, wdg, 0.0), axis=0, keepdims=True)
            if blk < 7:
                rsub = key[lo:lo + 128, lo + 128:]
                mr = (rsub > cs) if strict else (rsub >= cs)
                part = jnp.concatenate([part, jnp.sum(
                    jnp.where(mr, 2.0, 0.0), axis=0, keepdims=True)], axis=1)
            if lo:
                part = jnp.concatenate(
                    [jnp.zeros((1, lo), jnp.float32), part], axis=1)
            row = part if row is None else row + part
        return jnp.sum(row)

    def value_step(i, obits):
        cand = obits | jnp.left_shift(1, 31 - i)
        cnt = count_cmp(cand ^ _MININT, False)
        return jnp.where(cnt >= float(KEEP), cand, obits)

    obits = jax.lax.fori_loop(0, 32, value_step, jnp.int32(0))
    t_key = obits ^ _MININT

    above = key > t_key
    tie = key == t_key
    r_f = float(KEEP) - count_cmp(t_key, True)

    tie_bf = jnp.where(tie, 1.0, 0.0).astype(jnp.bfloat16)
    l_bf = jnp.where(jj < ii, 1.0, 0.0).astype(jnp.bfloat16)
    wc = jax.lax.dot_general(
        l_bf, tie_bf, (((1,), (0,)), ((), ())),
        preferred_element_type=jnp.float32)
    rc = wc[SEQ - 1:SEQ, :] + tie[SEQ - 1:SEQ, :].astype(jnp.float32)
    inc = rc
    s = 1
    while s < SEQ:
        inc = inc + jnp.concatenate(
            [jnp.zeros((1, s), jnp.float32), inc[:, :SEQ - s]], axis=1)
        s *= 2
    row_off = inc - rc

    thr_row = r_f - row_off
    keep_t = above | (tie & (wc < thr_row))
    at = jnp.where(keep_t, ew, 0.0)

    deg = jnp.sum(at, axis=1, keepdims=True) + 1.0
    dis = deg ** -0.5
    dis = jnp.where(jnp.isinf(dis), 0.0, dis)

    xt = jax.lax.dot_general(
        xs, w_ref[...], (((1,), (1,)), ((), ())),
        preferred_element_type=jnp.float32)

    sx = dis * xt
    y = jax.lax.dot_general(
        at, sx, (((1,), (0,)), ((), ())),
        preferred_element_type=jnp.float32)

    out_ref[...] = dis * y + (dis * dis) * xt + b_ref[...]


def kernel(x, W, b):
    xs = x.reshape(SEQ, DIM)
    b2 = b.reshape(1, DIM)
    out = pl.pallas_call(
        _gcn_kernel,
        out_shape=jax.ShapeDtypeStruct((SEQ, DIM), jnp.float32),
    )(xs, W, b2)
    return out[None, :, :]


# speedup vs baseline: 1.1779x; 1.0066x over previous
import jax
import jax.numpy as jnp
from jax.experimental import pallas as pl

SEQ = 1024
DIM = 128
KEEP = int(0.3 * SEQ * SEQ)

_MININT = -(2**31)


def _gcn_kernel(x_ref, w_ref, b_ref, out_ref):
    xs = x_ref[...]

    ew = jax.lax.dot_general(
        xs, xs, (((1,), (1,)), ((), ())),
        preferred_element_type=jnp.float32)
    ii = jax.lax.broadcasted_iota(jnp.int32, (SEQ, SEQ), 0)
    jj = jax.lax.broadcasted_iota(jnp.int32, (SEQ, SEQ), 1)
    ew = ew - jnp.where(ii == jj, 1.0, 0.0).astype(jnp.float32)

    ibits = jax.lax.bitcast_convert_type(ew, jnp.int32)
    key = jnp.where(ibits >= 0, ibits, ibits ^ 0x7FFFFFFF)

    ud = jax.lax.broadcasted_iota(jnp.int32, (128, 128), 0)
    vd = jax.lax.broadcasted_iota(jnp.int32, (128, 128), 1)
    wdg = jnp.where(vd > ud, 2.0,
                    jnp.where(vd == ud, 1.0, 0.0)).astype(jnp.float32)

    def count_cmp(cs, strict):
        row = None
        for blk in range(8):
            lo = 128 * blk
            dsub = key[lo:lo + 128, lo:lo + 128]
            md = (dsub > cs) if strict else (dsub >= cs)
            part = jnp.sum(jnp.where(md, wdg, 0.0), axis=0, keepdims=True)
            if blk < 7:
                rsub = key[lo:lo + 128, lo + 128:]
                mr = (rsub > cs) if strict else (rsub >= cs)
                part = jnp.concatenate([part, jnp.sum(
                    jnp.where(mr, 2.0, 0.0), axis=0, keepdims=True)], axis=1)
            if lo:
                part = jnp.concatenate(
                    [jnp.zeros((1, lo), jnp.float32), part], axis=1)
            row = part if row is None else row + part
        return jnp.sum(row)

    def value_step(i, state):
        obits, n_ge = state
        cand = obits | jnp.left_shift(1, 31 - i)
        cnt = count_cmp(cand ^ _MININT, False)
        ok = cnt >= float(KEEP)
        return (jnp.where(ok, cand, obits), jnp.where(ok, cnt, n_ge))

    obits, n_ge = jax.lax.fori_loop(
        0, 32, value_step, (jnp.int32(0), jnp.float32(SEQ * SEQ)))
    t_key = obits ^ _MININT

    above = key > t_key
    tie = key == t_key

    tie_bf = jnp.where(tie, 1.0, 0.0).astype(jnp.bfloat16)
    l_bf = jnp.where(jj < ii, 1.0, 0.0).astype(jnp.bfloat16)
    half = SEQ // 2
    wc = jnp.concatenate([
        jax.lax.dot_general(
            l_bf[:half], tie_bf, (((1,), (0,)), ((), ())),
            preferred_element_type=jnp.float32),
        jax.lax.dot_general(
            l_bf[half:], tie_bf, (((1,), (0,)), ((), ())),
            preferred_element_type=jnp.float32)], axis=0)
    rc = wc[SEQ - 1:SEQ, :] + tie[SEQ - 1:SEQ, :].astype(jnp.float32)
    r_f = float(KEEP) - n_ge + jnp.sum(rc)
    inc = rc
    s = 1
    while s < SEQ:
        inc = inc + jnp.concatenate(
            [jnp.zeros((1, s), jnp.float32), inc[:, :SEQ - s]], axis=1)
        s *= 2
    row_off = inc - rc

    thr_row = r_f - row_off
    keep_t = above | (tie & (wc < thr_row))
    at = jnp.where(keep_t, ew, 0.0)

    deg = jnp.sum(at, axis=1, keepdims=True) + 1.0
    dis = deg ** -0.5
    dis = jnp.where(jnp.isinf(dis), 0.0, dis)

    xt = jax.lax.dot_general(
        xs, w_ref[...], (((1,), (1,)), ((), ())),
        preferred_element_type=jnp.float32)

    sx = dis * xt
    y = jax.lax.dot_general(
        at, sx, (((1,), (0,)), ((), ())),
        preferred_element_type=jnp.float32)

    out_ref[...] = dis * y + (dis * dis) * xt + b_ref[...]


def kernel(x, W, b):
    xs = x.reshape(SEQ, DIM)
    b2 = b.reshape(1, DIM)
    out = pl.pallas_call(
        _gcn_kernel,
        out_shape=jax.ShapeDtypeStruct((SEQ, DIM), jnp.float32),
    )(xs, W, b2)
    return out[None, :, :]
